# baseline (speedup 1.0000x reference)
import functools

import jax
import jax.numpy as jnp
from jax import lax
from jax.experimental import pallas as pl
from jax.experimental.pallas import tpu as pltpu
from jax.experimental.pallas import tpu_sc as plsc

NUM_EMB = 1000000
D = 32
B = 4096
HIST = 200

NC = 2
NS = 16
NW = NC * NS
B_PER_W = B // NW

SPLITS = ((0, 104), (104, 96))

GROUPS = tuple(range(0, HIST - 16, 16)) + (HIST - 16,)


def _body(x_hbm, table_hbm, out_hbm, idx_v, rows_v, outb_v, sem):
    wid = lax.axis_index("s") * NC + lax.axis_index("c")
    pltpu.sync_copy(x_hbm.at[wid], idx_v)
    iot = lax.iota(jnp.int32, 16)

    def item(i, _):
        ibase = pl.multiple_of(i * HIST, 8)
        for off, ln in SPLITS:
            pltpu.async_copy(
                table_hbm.at[idx_v.at[pl.ds(ibase + off, ln)]],
                rows_v.at[pl.ds(off, ln)],
                sem,
            ).wait()

        def col(d, _):
            dvec = jnp.full((16,), 0, jnp.int32) + d
            obase = pl.multiple_of(d * HIST, 8)
            for l0 in GROUPS:
                vals = plsc.load_gather(rows_v, [iot + l0, dvec])
                outb_v[pl.ds(obase + l0, 16)] = vals
            return 0

        lax.fori_loop(0, D, col, 0)
        pltpu.sync_copy(outb_v, out_hbm.at[wid * B_PER_W + i])
        return 0

    lax.fori_loop(0, B_PER_W, item, 0)


@functools.partial(
    pl.kernel,
    mesh=plsc.VectorSubcoreMesh(core_axis_name="c", subcore_axis_name="s"),
    compiler_params=pltpu.CompilerParams(
        use_tc_tiling_on_sc=False, needs_layout_passes=False
    ),
    out_type=jax.ShapeDtypeStruct((B, D * HIST), jnp.float32),
    scratch_types=[
        pltpu.VMEM((B_PER_W * HIST,), jnp.int32),
        pltpu.VMEM((HIST, D), jnp.float32),
        pltpu.VMEM((D * HIST,), jnp.float32),
        pltpu.SemaphoreType.DMA,
    ],
)
def _emb_kernel(x_hbm, table_hbm, out_hbm, idx_v, rows_v, outb_v, sem):
    _body(x_hbm, table_hbm, out_hbm, idx_v, rows_v, outb_v, sem)


def kernel(x, table):
    x_r = x.astype(jnp.int32).reshape(NW, B_PER_W * HIST)
    out = _emb_kernel(x_r, table)
    return out.reshape(B, D, HIST)


# speedup vs baseline: 1.1724x; 1.1724x over previous
import functools

import jax
import jax.numpy as jnp
from jax import lax
from jax.experimental import pallas as pl
from jax.experimental.pallas import tpu as pltpu
from jax.experimental.pallas import tpu_sc as plsc

NUM_EMB = 1000000
D = 32
B = 4096
HIST = 200

NC = 2
NS = 16
NW = NC * NS
B_PER_W = B // NW
C = 2
NCHUNK = B_PER_W // C
NPAIR = NCHUNK // 2

SPLITS = ((0, 104), (104, 96))

GROUPS = tuple(range(0, HIST - 16, 16)) + (HIST - 16,)


def _body(x_hbm, table_hbm, out_hbm, idx_v, rows0, rows1, outb0, outb1,
          gsem0, gsem1, osem0, osem1):
    wid = lax.axis_index("s") * NC + lax.axis_index("c")
    pltpu.sync_copy(x_hbm.at[wid], idx_v)
    iot = lax.iota(jnp.int32, 16)
    out_base = wid * B_PER_W

    def issue_gathers(c, rows_buf, sem):
        ibase = pl.multiple_of(c * (C * HIST), 8)
        for it in range(C):
            for off, ln in SPLITS:
                o = it * HIST + off
                pltpu.async_copy(
                    table_hbm.at[idx_v.at[pl.ds(ibase + o, ln)]],
                    rows_buf.at[pl.ds(o, ln)],
                    sem,
                )

    def drain_gathers(rows_buf, sem):
        pltpu.make_async_copy(
            table_hbm.at[pl.ds(0, C * HIST)], rows_buf, sem
        ).wait()

    def issue_out(c, outb_buf, sem):
        pltpu.async_copy(outb_buf, out_hbm.at[pl.ds(out_base + c * C, C)], sem)

    def drain_out(outb_buf, sem):
        pltpu.make_async_copy(
            outb_buf, out_hbm.at[pl.ds(out_base, C)], sem
        ).wait()

    def transpose(rows_buf, outb_buf):
        def col(d, _):
            dvec = iot * 0 + d
            obase = pl.multiple_of(d * HIST, 8)
            for it in range(C):
                for l0 in GROUPS:
                    vals = plsc.load_gather(
                        rows_buf, [iot + (it * HIST + l0), dvec]
                    )
                    outb_buf[it, pl.ds(obase + l0, 16)] = vals
            return 0

        lax.fori_loop(0, D, col, 0)

    issue_gathers(0, rows0, gsem0)

    def pair(k, _):
        ca = 2 * k
        cb = ca + 1
        issue_gathers(cb, rows1, gsem1)
        drain_gathers(rows0, gsem0)

        @pl.when(k > 0)
        def _():
            drain_out(outb0, osem0)

        transpose(rows0, outb0)
        issue_out(ca, outb0, osem0)

        @pl.when(k < NPAIR - 1)
        def _():
            issue_gathers(ca + 2, rows0, gsem0)

        drain_gathers(rows1, gsem1)

        @pl.when(k > 0)
        def _():
            drain_out(outb1, osem1)

        transpose(rows1, outb1)
        issue_out(cb, outb1, osem1)
        return 0

    lax.fori_loop(0, NPAIR, pair, 0)
    drain_out(outb0, osem0)
    drain_out(outb1, osem1)


@functools.partial(
    pl.kernel,
    mesh=plsc.VectorSubcoreMesh(core_axis_name="c", subcore_axis_name="s"),
    compiler_params=pltpu.CompilerParams(
        use_tc_tiling_on_sc=False, needs_layout_passes=False
    ),
    out_type=jax.ShapeDtypeStruct((B, D * HIST), jnp.float32),
    scratch_types=[
        pltpu.VMEM((B_PER_W * HIST,), jnp.int32),
        pltpu.VMEM((C * HIST, D), jnp.float32),
        pltpu.VMEM((C * HIST, D), jnp.float32),
        pltpu.VMEM((C, D * HIST), jnp.float32),
        pltpu.VMEM((C, D * HIST), jnp.float32),
        pltpu.SemaphoreType.DMA,
        pltpu.SemaphoreType.DMA,
        pltpu.SemaphoreType.DMA,
        pltpu.SemaphoreType.DMA,
    ],
)
def _emb_kernel(x_hbm, table_hbm, out_hbm, idx_v, rows0, rows1, outb0, outb1,
                gsem0, gsem1, osem0, osem1):
    _body(x_hbm, table_hbm, out_hbm, idx_v, rows0, rows1, outb0, outb1,
          gsem0, gsem1, osem0, osem1)


def kernel(x, table):
    x_r = x.astype(jnp.int32).reshape(NW, B_PER_W * HIST)
    out = _emb_kernel(x_r, table)
    return out.reshape(B, D, HIST)


# speedup vs baseline: 1.7540x; 1.4961x over previous
import functools

import jax
import jax.numpy as jnp
from jax import lax
from jax.experimental import pallas as pl
from jax.experimental.pallas import tpu as pltpu
from jax.experimental.pallas import tpu_sc as plsc

NUM_EMB = 1000000
D = 32
B = 4096
HIST = 200
HISTP = 201

NC = 2
NS = 16
NW = NC * NS
B_PER_W = B // NW
C = 2
NCHUNK = B_PER_W // C
NPAIR = NCHUNK // 2

SPLITS = ((0, 104), (104, 96))


def _body(x_hbm, table_hbm, out_hbm, idx_v, rows0, rows1, outb0, outb1,
          gsem0, gsem1, osem0, osem1):
    wid = lax.axis_index("s") * NC + lax.axis_index("c")
    pltpu.sync_copy(x_hbm.at[wid], idx_v)
    iot = lax.iota(jnp.int32, 16)
    out_base = wid * B_PER_W

    def issue_gathers(c, rows_buf, sem):
        ibase = pl.multiple_of(c * (C * HIST), 8)
        for it in range(C):
            for off, ln in SPLITS:
                o = it * HIST + off
                pltpu.async_copy(
                    table_hbm.at[idx_v.at[pl.ds(ibase + o, ln)]],
                    rows_buf.at[pl.ds(o, ln)],
                    sem,
                )

    def drain_gathers(rows_buf, sem):
        pltpu.make_async_copy(
            table_hbm.at[pl.ds(0, C * HIST)], rows_buf, sem
        ).wait()

    def issue_out(c, outb_buf, sem):
        pltpu.async_copy(
            outb_buf.at[pl.ds(0, C * D), pl.ds(0, HIST)],
            out_hbm.at[pl.ds((out_base + c * C) * D, C * D)],
            sem,
        )

    def drain_out(outb_buf, sem):
        pltpu.make_async_copy(
            outb_buf.at[pl.ds(0, C * D), pl.ds(0, HIST)],
            out_hbm.at[pl.ds(0, C * D)],
            sem,
        ).wait()

    def transpose(rows_buf, outb_buf):
        ridx = [
            (iot + it * D, iot + (16 + it * D)) for it in range(C)
        ]

        @plsc.parallel_loop(0, HIST, unroll=4)
        def _(l):
            cidx = iot * 0 + l
            for it in range(C):
                v0 = rows_buf[it * HIST + l, pl.ds(0, 16)]
                plsc.store_scatter(outb_buf, [ridx[it][0], cidx], v0)
                v1 = rows_buf[it * HIST + l, pl.ds(16, 16)]
                plsc.store_scatter(outb_buf, [ridx[it][1], cidx], v1)

    issue_gathers(0, rows0, gsem0)

    def pair(k, _):
        ca = 2 * k
        cb = ca + 1
        issue_gathers(cb, rows1, gsem1)
        drain_gathers(rows0, gsem0)

        @pl.when(k > 0)
        def _():
            drain_out(outb0, osem0)

        transpose(rows0, outb0)
        issue_out(ca, outb0, osem0)

        @pl.when(k < NPAIR - 1)
        def _():
            issue_gathers(ca + 2, rows0, gsem0)

        drain_gathers(rows1, gsem1)

        @pl.when(k > 0)
        def _():
            drain_out(outb1, osem1)

        transpose(rows1, outb1)
        issue_out(cb, outb1, osem1)
        return 0

    lax.fori_loop(0, NPAIR, pair, 0)
    drain_out(outb0, osem0)
    drain_out(outb1, osem1)


@functools.partial(
    pl.kernel,
    mesh=plsc.VectorSubcoreMesh(core_axis_name="c", subcore_axis_name="s"),
    compiler_params=pltpu.CompilerParams(
        use_tc_tiling_on_sc=False, needs_layout_passes=False
    ),
    out_type=jax.ShapeDtypeStruct((B * D, HIST), jnp.float32),
    scratch_types=[
        pltpu.VMEM((B_PER_W * HIST,), jnp.int32),
        pltpu.VMEM((C * HIST, D), jnp.float32),
        pltpu.VMEM((C * HIST, D), jnp.float32),
        pltpu.VMEM((C * D, HISTP), jnp.float32),
        pltpu.VMEM((C * D, HISTP), jnp.float32),
        pltpu.SemaphoreType.DMA,
        pltpu.SemaphoreType.DMA,
        pltpu.SemaphoreType.DMA,
        pltpu.SemaphoreType.DMA,
    ],
)
def _emb_kernel(x_hbm, table_hbm, out_hbm, idx_v, rows0, rows1, outb0, outb1,
                gsem0, gsem1, osem0, osem1):
    _body(x_hbm, table_hbm, out_hbm, idx_v, rows0, rows1, outb0, outb1,
          gsem0, gsem1, osem0, osem1)


def kernel(x, table):
    x_r = x.astype(jnp.int32).reshape(NW, B_PER_W * HIST)
    out = _emb_kernel(x_r, table)
    return out.reshape(B, D, HIST)


# speedup vs baseline: 2.6002x; 1.4825x over previous
import functools

import jax
import jax.numpy as jnp
from jax import lax
from jax.experimental import pallas as pl
from jax.experimental.pallas import tpu as pltpu
from jax.experimental.pallas import tpu_sc as plsc

NUM_EMB = 1000000
D = 32
B = 4096
HIST = 200
HISTP = 201

NC = 2
NS = 16
NW = NC * NS
B_PER_W = B // NW
LT = HIST // 8
HC = 512
NPAIR = LT
STGP = 517


def _body(x_hbm, table_hbm, out_hbm, idx_v, gidx, rows0, rows1, stg0, stg1,
          gsem0, gsem1, osem0, osem1):
    wid = lax.axis_index("s") * NC + lax.axis_index("c")
    pltpu.sync_copy(x_hbm.at[wid], idx_v.at[:, pl.ds(0, HIST)])
    iot = lax.iota(jnp.int32, 16)
    bases = [k * 16 + iot for k in range(8)]
    dvec0 = iot * STGP
    dvec1 = dvec0 + 16 * STGP

    def rearrange(lt, _):
        for li in range(8):
            col = lt * 8 + li
            cvec = iot * 0 + col
            for k in range(8):
                vals = plsc.load_gather(idx_v, [bases[k], cvec])
                gidx[lt, pl.ds(li * 128 + k * 16, 16)] = vals
        return 0

    lax.fori_loop(0, LT, rearrange, 0)

    def issue_gathers(hc, rows_buf, sem):
        lt = hc >> 1
        off = (hc & 1) * HC
        for g in range(4):
            pltpu.async_copy(
                table_hbm.at[gidx.at[lt, pl.ds(off + g * 128, 128)]],
                rows_buf.at[pl.ds(g * 128, 128)],
                sem,
            )

    def drain_gathers(rows_buf, sem):
        pltpu.make_async_copy(
            table_hbm.at[pl.ds(0, HC)], rows_buf, sem
        ).wait()

    def issue_out(hc, stg_buf, sem):
        lt = hc >> 1
        off = (hc & 1) * HC
        pltpu.async_copy(
            stg_buf.at[:, pl.ds(0, HC)],
            out_hbm.at[:, lt, wid, pl.ds(off, HC)],
            sem,
        )

    def drain_out(stg_buf, sem):
        pltpu.make_async_copy(
            stg_buf.at[:, pl.ds(0, HC)],
            out_hbm.at[:, 0, 0, pl.ds(0, HC)],
            sem,
        ).wait()

    def transpose(rows_buf, stg_buf):
        @plsc.parallel_loop(0, HC, unroll=4)
        def _(g):
            gvec = iot * 0 + g
            v0 = rows_buf[g, pl.ds(0, 16)]
            plsc.store_scatter(stg_buf, [iot, gvec], v0)
            v1 = rows_buf[g, pl.ds(16, 16)]
            plsc.store_scatter(stg_buf, [iot + 16, gvec], v1)

    issue_gathers(0, rows0, gsem0)

    def pair(k, _):
        ha = 2 * k
        hb = ha + 1
        issue_gathers(hb, rows1, gsem1)
        drain_gathers(rows0, gsem0)

        @pl.when(k > 0)
        def _():
            drain_out(stg0, osem0)

        transpose(rows0, stg0)
        issue_out(ha, stg0, osem0)

        @pl.when(k < NPAIR - 1)
        def _():
            issue_gathers(ha + 2, rows0, gsem0)

        drain_gathers(rows1, gsem1)

        @pl.when(k > 0)
        def _():
            drain_out(stg1, osem1)

        transpose(rows1, stg1)
        issue_out(hb, stg1, osem1)
        return 0

    lax.fori_loop(0, NPAIR, pair, 0)
    drain_out(stg0, osem0)
    drain_out(stg1, osem1)


@functools.partial(
    pl.kernel,
    mesh=plsc.VectorSubcoreMesh(core_axis_name="c", subcore_axis_name="s"),
    compiler_params=pltpu.CompilerParams(
        use_tc_tiling_on_sc=False, needs_layout_passes=False
    ),
    out_type=jax.ShapeDtypeStruct((D, LT, NW, 1024), jnp.float32),
    scratch_types=[
        pltpu.VMEM((B_PER_W, HISTP), jnp.int32),
        pltpu.VMEM((LT, 1024), jnp.int32),
        pltpu.VMEM((HC, D), jnp.float32),
        pltpu.VMEM((HC, D), jnp.float32),
        pltpu.VMEM((D, STGP), jnp.float32),
        pltpu.VMEM((D, STGP), jnp.float32),
        pltpu.SemaphoreType.DMA,
        pltpu.SemaphoreType.DMA,
        pltpu.SemaphoreType.DMA,
        pltpu.SemaphoreType.DMA,
    ],
)
def _emb_kernel(x_hbm, table_hbm, out_hbm, idx_v, gidx, rows0, rows1,
                stg0, stg1, gsem0, gsem1, osem0, osem1):
    _body(x_hbm, table_hbm, out_hbm, idx_v, gidx, rows0, rows1, stg0, stg1,
          gsem0, gsem1, osem0, osem1)


def kernel(x, table):
    x_r = x.astype(jnp.int32).reshape(NW, B_PER_W, HIST)
    out4 = _emb_kernel(x_r, table)
    out5 = out4.reshape(D, LT, NW, 8, 128)
    return out5.transpose(2, 4, 0, 1, 3).reshape(B, D, HIST)
